# Initial kernel scaffold; baseline (speedup 1.0000x reference)
import jax
import jax.numpy as jnp
from jax.experimental import pallas as pl


def kernel(latents, embedding):
    raise NotImplementedError("write your pallas kernel here")


# speedup vs baseline: 1.3629x; 1.3629x over previous
import jax
import jax.numpy as jnp
from jax.experimental import pallas as pl

_NUM_EMBEDDINGS = 1024
_EMBEDDING_DIM = 32
_COMMITMENT_COST = 0.25


def _vq_kernel(lat_ref, emb_ref, f2_ref, e2_ref, out_ref, loss_ref):
    b = pl.program_id(0)
    lat = lat_ref[0]
    flat = lat.T
    emb = emb_ref[...]
    f2 = f2_ref[0]
    e2 = e2_ref[...]
    d = (f2 + e2) - 2.0 * jnp.dot(flat, emb.T,
                                  preferred_element_type=jnp.float32)
    iota = jax.lax.broadcasted_iota(jnp.int32, d.shape, 1)
    dmin = jnp.min(d, axis=1, keepdims=True)
    idx = jnp.min(jnp.where(d == dmin, iota, d.shape[1]), axis=1)
    onehot = (iota == idx[:, None]).astype(jnp.float32)
    q = jnp.dot(onehot, emb, preferred_element_type=jnp.float32)
    diff = flat - q
    sq = jnp.sum(diff * diff).reshape(1, 1)

    @pl.when(b == 0)
    def _init():
        loss_ref[...] = jnp.zeros((1, 1), jnp.float32)

    loss_ref[...] += sq
    out_ref[0] = q.T


def kernel(latents, embedding):
    B, C, H, W = latents.shape
    K = embedding.shape[0]
    HW = H * W
    lat3 = latents.reshape(B, C, HW)
    latents_r = jnp.moveaxis(latents, 1, -1)
    flat_all = latents_r.reshape(-1, C)
    f2_all = jnp.sum(flat_all ** 2, axis=1, keepdims=True).reshape(B, HW, 1)
    e2_all = jnp.sum(embedding ** 2, axis=1).reshape(1, K)

    out, loss_sum = pl.pallas_call(
        _vq_kernel,
        grid=(B,),
        in_specs=[
            pl.BlockSpec((1, C, HW), lambda b: (b, 0, 0)),
            pl.BlockSpec((K, C), lambda b: (0, 0)),
            pl.BlockSpec((1, HW, 1), lambda b: (b, 0, 0)),
            pl.BlockSpec((1, K), lambda b: (0, 0)),
        ],
        out_specs=[
            pl.BlockSpec((1, C, HW), lambda b: (b, 0, 0)),
            pl.BlockSpec((1, 1), lambda b: (0, 0)),
        ],
        out_shape=[
            jax.ShapeDtypeStruct((B, C, HW), jnp.float32),
            jax.ShapeDtypeStruct((1, 1), jnp.float32),
        ],
    )(lat3, embedding, f2_all, e2_all)
    n_elems = B * C * HW
    vq_loss = (1.0 + _COMMITMENT_COST) * loss_sum[0, 0] / n_elems
    return out.reshape(B, C, H, W), vq_loss
